# Initial kernel scaffold; baseline (speedup 1.0000x reference)
import jax
import jax.numpy as jnp
from jax.experimental import pallas as pl


def kernel(x, W, bias):
    raise NotImplementedError("write your pallas kernel here")


# speedup vs baseline: 1.2239x; 1.2239x over previous
import functools

import jax
import jax.numpy as jnp
from jax import lax
from jax.experimental import pallas as pl
from jax.experimental.pallas import tpu as pltpu
from jax.experimental.pallas import tpu_sc as plsc

F = 26
B = 16384
V = 100000
NC = 2
NS = 16
L = 16
CHUNK = 2048

_mesh = plsc.VectorSubcoreMesh(
    core_axis_name="c", subcore_axis_name="s", num_cores=NC, num_subcores=NS
)


@functools.partial(
    pl.kernel,
    out_type=jax.ShapeDtypeStruct((F * B,), jnp.float32),
    mesh=_mesh,
    scratch_types=[
        pltpu.VMEM((V,), jnp.float32),
        pltpu.VMEM((CHUNK,), jnp.int32),
        pltpu.VMEM((CHUNK,), jnp.float32),
    ],
    compiler_params=pltpu.CompilerParams(
        use_tc_tiling_on_sc=False, needs_layout_passes=False
    ),
)
def _gather_fields(w_hbm, idx_hbm, part_hbm, tbl_v, idx_v, val_v):
    c = lax.axis_index("c")
    s = lax.axis_index("s")
    field = s * NC + c

    @pl.when(field < F)
    def _():
        pltpu.sync_copy(w_hbm.at[pl.ds(field * V, V)], tbl_v)

        def chunk_body(ci, carry):
            base = field * B + ci * CHUNK
            pltpu.sync_copy(idx_hbm.at[pl.ds(base, CHUNK)], idx_v)

            def gather_body(i, carry2):
                off = i * L
                val_v[pl.ds(off, L)] = plsc.load_gather(
                    tbl_v, [idx_v[pl.ds(off, L)]]
                )
                return carry2

            lax.fori_loop(0, CHUNK // L, gather_body, 0, unroll=8)
            pltpu.sync_copy(val_v, part_hbm.at[pl.ds(base, CHUNK)])
            return carry

        lax.fori_loop(0, B // CHUNK, chunk_body, 0)


def _finish_body(p_ref, b_ref, o_ref):
    total = jnp.sum(p_ref[...], axis=0, keepdims=True) + b_ref[...]
    o_ref[...] = jax.nn.sigmoid(total)


_finish = pl.pallas_call(
    _finish_body,
    out_shape=jax.ShapeDtypeStruct((1, B), jnp.float32),
)


def kernel(x, W, bias):
    idx = x.astype(jnp.int32).T.reshape(F * B)
    w_flat = W.reshape(F * V)
    partials = _gather_fields(w_flat, idx)
    out = _finish(partials.reshape(F, B), bias.reshape(1, 1))
    return out.reshape(B)


# speedup vs baseline: 1.2339x; 1.0082x over previous
import functools

import jax
import jax.numpy as jnp
from jax import lax
from jax.experimental import pallas as pl
from jax.experimental.pallas import tpu as pltpu
from jax.experimental.pallas import tpu_sc as plsc

F = 26
B = 16384
V = 100000
NC = 2
NS = 16
L = 16
CHUNK = 2048

_mesh = plsc.VectorSubcoreMesh(
    core_axis_name="c", subcore_axis_name="s", num_cores=NC, num_subcores=NS
)


@functools.partial(
    pl.kernel,
    out_type=jax.ShapeDtypeStruct((F * B,), jnp.float32),
    mesh=_mesh,
    scratch_types=[
        pltpu.VMEM((V,), jnp.float32),
        pltpu.VMEM((CHUNK,), jnp.int32),
        pltpu.VMEM((CHUNK,), jnp.float32),
    ],
    compiler_params=pltpu.CompilerParams(
        use_tc_tiling_on_sc=False, needs_layout_passes=False
    ),
)
def _gather_fields(w_hbm, idx_hbm, part_hbm, tbl_v, idx_v, val_v):
    c = lax.axis_index("c")
    s = lax.axis_index("s")
    field = s * NC + c

    @pl.when(field < F)
    def _():
        pltpu.sync_copy(w_hbm.at[0, pl.ds(field * V, V)], tbl_v)

        def chunk_body(ci, carry):
            base = field * B + ci * CHUNK
            pltpu.sync_copy(idx_hbm.at[pl.ds(base, CHUNK)], idx_v)

            def gather_body(i, carry2):
                off = i * L
                val_v[pl.ds(off, L)] = plsc.load_gather(
                    tbl_v, [idx_v[pl.ds(off, L)]]
                )
                return carry2

            lax.fori_loop(0, CHUNK // L, gather_body, 0, unroll=8)
            pltpu.sync_copy(val_v, part_hbm.at[pl.ds(base, CHUNK)])
            return carry

        lax.fori_loop(0, B // CHUNK, chunk_body, 0)


def _finish_body(p_ref, b_ref, o_ref):
    total = jnp.sum(p_ref[...], axis=0, keepdims=True) + b_ref[...]
    o_ref[...] = jax.nn.sigmoid(total)


_finish = pl.pallas_call(
    _finish_body,
    out_shape=jax.ShapeDtypeStruct((1, B), jnp.float32),
)


def kernel(x, W, bias):
    idx = x.astype(jnp.int32).T.reshape(F * B)
    partials = _gather_fields(W.T, idx)
    out = _finish(partials.reshape(F, B), bias.reshape(1, 1))
    return out.reshape(B)
